# Initial kernel scaffold; baseline (speedup 1.0000x reference)
import jax
import jax.numpy as jnp
from jax.experimental import pallas as pl


def kernel(x, edge_index, W1, beta0, beta1):
    raise NotImplementedError("write your pallas kernel here")


# speedup vs baseline: 3.1907x; 3.1907x over previous
import jax
import jax.numpy as jnp
from jax import lax
from jax.experimental import pallas as pl
from jax.experimental.pallas import tpu as pltpu
from jax.experimental.pallas import tpu_sc as plsc

_LANES = 16
_CHUNK = 128
_N_TILES = 32


def _norm_body(x_ref, xn_ref, nrm_ref):
    x = x_ref[...]
    nrm = jnp.sqrt(jnp.sum(x * x, axis=1, keepdims=True))
    xn_ref[...] = x / (nrm + 1e-12)
    nrm_ref[...] = nrm


def _normalize(x, block_rows):
    n, f = x.shape
    return pl.pallas_call(
        _norm_body,
        grid=(n // block_rows,),
        in_specs=[pl.BlockSpec((block_rows, f), lambda i: (i, 0))],
        out_specs=[
            pl.BlockSpec((block_rows, f), lambda i: (i, 0)),
            pl.BlockSpec((block_rows, 1), lambda i: (i, 0)),
        ],
        out_shape=[
            jax.ShapeDtypeStruct((n, f), jnp.float32),
            jax.ShapeDtypeStruct((n, 1), jnp.float32),
        ],
    )(x)


def _mid_body(out0_ref, s_ref, w_ref, xn1_ref, nrm1_ref):
    s = jnp.sum(s_ref[...], axis=1, keepdims=True) + 1e-12
    h = jnp.tanh(out0_ref[...] / s)
    y = jnp.dot(h, w_ref[...], preferred_element_type=jnp.float32)
    n1 = jnp.sqrt(jnp.sum(y * y, axis=1, keepdims=True))
    xn1_ref[...] = y / (n1 + 1e-12)
    nrm1_ref[...] = n1


def _mid(out0, s_t, w1, block_rows):
    n, f = out0.shape
    t = s_t.shape[1]
    c = w1.shape[1]
    return pl.pallas_call(
        _mid_body,
        grid=(n // block_rows,),
        in_specs=[
            pl.BlockSpec((block_rows, f), lambda i: (i, 0)),
            pl.BlockSpec((block_rows, t), lambda i: (i, 0)),
            pl.BlockSpec((f, c), lambda i: (0, 0)),
        ],
        out_specs=[
            pl.BlockSpec((block_rows, c), lambda i: (i, 0)),
            pl.BlockSpec((block_rows, 1), lambda i: (i, 0)),
        ],
        out_shape=[
            jax.ShapeDtypeStruct((n, c), jnp.float32),
            jax.ShapeDtypeStruct((n, 1), jnp.float32),
        ],
    )(out0, s_t, w1)


def _fin_body(outa_ref, outb_ref, s_ref, o_ref):
    s = jnp.sum(s_ref[...], axis=1, keepdims=True) + 1e-12
    o_ref[...] = (outa_ref[...] + outb_ref[...]) / s


def _final(outa, outb, s_t, block_rows):
    n, c = outa.shape
    t = s_t.shape[1]
    return pl.pallas_call(
        _fin_body,
        grid=(n // block_rows,),
        in_specs=[
            pl.BlockSpec((block_rows, c), lambda i: (i, 0)),
            pl.BlockSpec((block_rows, c), lambda i: (i, 0)),
            pl.BlockSpec((block_rows, t), lambda i: (i, 0)),
        ],
        out_specs=pl.BlockSpec((block_rows, c), lambda i: (i, 0)),
        out_shape=jax.ShapeDtypeStruct((n, c), jnp.float32),
    )(outa, outb, s_t)


def _edge_weights(f_dim, e_pad, sp):
    n_rows = e_pad // _CHUNK
    rows_per_tile = n_rows // _N_TILES
    kf = f_dim // _LANES
    mesh = plsc.VectorSubcoreMesh(core_axis_name="c", subcore_axis_name="s")

    def body(xn_ref, scale_ref, src_ref, dst_ref, beta_ref, m_ref, spart_ref,
             src_v, dst_v, rows_s, rows_d, cosb, m_v, s_loc, scale_v, beta_v,
             sem):
        wid = lax.axis_index("s") * 2 + lax.axis_index("c")
        pltpu.sync_copy(beta_ref, beta_v)
        pltpu.sync_copy(scale_ref, scale_v)
        beta_vec = beta_v[...]
        zeros16 = jnp.zeros((_LANES,), jnp.float32)
        lane0 = lax.iota(jnp.int32, _LANES) == 0

        def zero_step(i, carry):
            s_loc[pl.ds(i * _LANES, _LANES)] = zeros16
            return carry

        lax.fori_loop(0, sp // _LANES, zero_step, 0)

        def chunk_step(j, carry):
            r = wid * rows_per_tile + j
            pltpu.sync_copy(src_ref.at[pl.ds(r * _CHUNK, _CHUNK)], src_v)
            pltpu.sync_copy(dst_ref.at[pl.ds(r * _CHUNK, _CHUNK)], dst_v)
            pltpu.async_copy(xn_ref.at[src_v], rows_s, sem).wait()
            pltpu.async_copy(xn_ref.at[dst_v], rows_d, sem).wait()

            def edge_step(e, ecarry):
                acc = rows_s[e, pl.ds(0, _LANES)] * rows_d[e, pl.ds(0, _LANES)]
                for k in range(1, kf):
                    sl = pl.ds(k * _LANES, _LANES)
                    acc = acc + rows_s[e, sl] * rows_d[e, sl]
                cval = jnp.broadcast_to(jnp.sum(acc), (_LANES,))
                eidx = jnp.broadcast_to(e, (_LANES,)).astype(jnp.int32)
                plsc.store_scatter(cosb, [eidx], cval, mask=lane0)
                return ecarry

            lax.fori_loop(0, _CHUNK, edge_step, 0)

            for g in range(_CHUNK // _LANES):
                sl = pl.ds(g * _LANES, _LANES)
                w_v = jnp.exp(beta_vec * cosb[sl])
                sc = plsc.load_gather(scale_v, [src_v[sl]])
                m_v[sl] = w_v * sc
                plsc.addupdate_scatter(s_loc, [dst_v[sl]], w_v)
            pltpu.sync_copy(m_v, m_ref.at[pl.ds(r * _CHUNK, _CHUNK)])
            return carry

        lax.fori_loop(0, rows_per_tile, chunk_step, 0)
        pltpu.sync_copy(s_loc, spart_ref.at[wid])

    return pl.kernel(
        body,
        out_type=(
            jax.ShapeDtypeStruct((e_pad,), jnp.float32),
            jax.ShapeDtypeStruct((_N_TILES, sp), jnp.float32),
        ),
        mesh=mesh,
        compiler_params=pltpu.CompilerParams(needs_layout_passes=False),
        scratch_types=(
            pltpu.VMEM((_CHUNK,), jnp.int32),
            pltpu.VMEM((_CHUNK,), jnp.int32),
            pltpu.VMEM((_CHUNK, f_dim), jnp.float32),
            pltpu.VMEM((_CHUNK, f_dim), jnp.float32),
            pltpu.VMEM((_CHUNK,), jnp.float32),
            pltpu.VMEM((_CHUNK,), jnp.float32),
            pltpu.VMEM((sp,), jnp.float32),
            pltpu.VMEM((sp,), jnp.float32),
            pltpu.VMEM((_LANES,), jnp.float32),
            pltpu.SemaphoreType.DMA,
        ),
    )


def _message_scatter(fh, e_pad, np_acc, n_nodes, split):
    n_rows = e_pad // _CHUNK
    rows_per_tile = n_rows // (16 if split == "feat" else _N_TILES)
    jf = fh // _LANES
    mesh = plsc.VectorSubcoreMesh(core_axis_name="c", subcore_axis_name="s")

    def body(xcat_ref, src_ref, dst2_ref, m_ref, out_ref,
             src_v, dst_i, m_v, rows, zer, acc, sem):
        c = lax.axis_index("c")
        s_id = lax.axis_index("s")
        zeros16 = jnp.zeros((_LANES,), jnp.float32)

        def zrow(e, carry):
            for j in range(jf):
                zer[e, pl.ds(j * _LANES, _LANES)] = zeros16
            return carry

        lax.fori_loop(0, _CHUNK, zrow, 0)

        rows_zero_per_tile = np_acc // 16
        nz = rows_zero_per_tile // _CHUNK
        base = s_id * rows_zero_per_tile

        def zcopy(i, carry):
            pltpu.sync_copy(zer, acc.at[pl.ds(base + i * _CHUNK, _CHUNK)])
            return carry

        lax.fori_loop(0, nz, zcopy, 0)
        plsc.subcore_barrier()

        def chunk_step(j, carry):
            if split == "feat":
                r = s_id * rows_per_tile + j
            else:
                r = (s_id * 2 + c) * rows_per_tile + j
            pltpu.sync_copy(src_ref.at[pl.ds(r * _CHUNK, _CHUNK)], src_v)
            pltpu.sync_copy(dst2_ref.at[pl.ds(r, 1)], dst_i)
            pltpu.sync_copy(m_ref.at[pl.ds(r * _CHUNK, _CHUNK)], m_v)
            if split == "feat":
                coff = c * n_nodes
                for g in range(_CHUNK // _LANES):
                    sl = pl.ds(g * _LANES, _LANES)
                    src_v[sl] = src_v[sl] + coff
            pltpu.async_copy(xcat_ref.at[src_v], rows, sem).wait()

            def scale_e(e, ecarry):
                eidx = jnp.broadcast_to(e, (_LANES,)).astype(jnp.int32)
                mv = plsc.load_gather(m_v, [eidx])
                for j2 in range(jf):
                    sl = pl.ds(j2 * _LANES, _LANES)
                    rows[e, sl] = rows[e, sl] * mv
                return ecarry

            lax.fori_loop(0, _CHUNK, scale_e, 0)
            pltpu.sync_copy(rows, acc.at[dst_i.at[0]], add=True)
            return carry

        lax.fori_loop(0, rows_per_tile, chunk_step, 0)
        plsc.subcore_barrier()

        obase = c * np_acc + base

        def ocopy(i, carry):
            pltpu.sync_copy(acc.at[pl.ds(base + i * _CHUNK, _CHUNK)],
                            out_ref.at[pl.ds(obase + i * _CHUNK, _CHUNK)])
            return carry

        lax.fori_loop(0, nz, ocopy, 0)

    return pl.kernel(
        body,
        out_type=jax.ShapeDtypeStruct((2 * np_acc, fh), jnp.float32),
        mesh=mesh,
        compiler_params=pltpu.CompilerParams(needs_layout_passes=False),
        scratch_types=(
            pltpu.VMEM((_CHUNK,), jnp.int32),
            pltpu.VMEM((1, _CHUNK), jnp.int32),
            pltpu.VMEM((_CHUNK,), jnp.float32),
            pltpu.VMEM((_CHUNK, fh), jnp.float32),
            pltpu.VMEM((_CHUNK, fh), jnp.float32),
            pltpu.VMEM_SHARED((np_acc, fh), jnp.float32),
            pltpu.SemaphoreType.DMA,
        ),
    )


def _agnn_layer(xn, nrm, src_p, dst_p, dst2, beta, n, e_pad, sp, np_acc,
                split):
    f = xn.shape[1]
    xn_pad = jnp.concatenate([xn, jnp.zeros((1, f), jnp.float32)])
    scale_pad = jnp.concatenate(
        [nrm[:, 0], jnp.zeros((sp - n,), jnp.float32)])
    beta_arr = jnp.zeros((_LANES,), jnp.float32) + beta.astype(jnp.float32)
    m_e, s_part = _edge_weights(f, e_pad, sp)(
        xn_pad, scale_pad, src_p, dst_p, beta_arr)
    s_t = s_part[:, :n].T
    if split == "feat":
        xcat = jnp.concatenate([xn[:, : f // 2], xn[:, f // 2:]], axis=0)
        out_cat = _message_scatter(f // 2, e_pad, np_acc, n, split)(
            xcat, src_p, dst2, m_e)
        out = jnp.concatenate(
            [out_cat[:n], out_cat[np_acc:np_acc + n]], axis=1)
        return (out,), s_t
    out_cat = _message_scatter(f, e_pad, np_acc, n, split)(
        xn, src_p, dst2, m_e)
    return (out_cat[:n], out_cat[np_acc:np_acc + n]), s_t


@jax.jit
def kernel(x, edge_index, W1, beta0, beta1):
    n, f = x.shape
    c_dim = W1.shape[1]
    e = edge_index.shape[1]
    e_pad = -(-e // (_N_TILES * _CHUNK)) * (_N_TILES * _CHUNK)
    sp = -(-(n + 1) // _LANES) * _LANES
    np_acc = -(-(n + 1) // (16 * _CHUNK)) * (16 * _CHUNK)
    block = 1000 if n % 1000 == 0 else 8

    src = edge_index[0].astype(jnp.int32)
    dst = edge_index[1].astype(jnp.int32)
    src_p = jnp.concatenate([src, jnp.zeros((e_pad - e,), jnp.int32)])
    dst_p = jnp.concatenate([dst, jnp.full((e_pad - e,), n, jnp.int32)])
    dst2 = dst_p.reshape(e_pad // _CHUNK, _CHUNK)

    xn, nrm = _normalize(x, block)
    (out0,), s_t0 = _agnn_layer(
        xn, nrm, src_p, dst_p, dst2, beta0, n, e_pad, sp, np_acc, "feat")

    xn1, nrm1 = _mid(out0, s_t0, W1, block)
    xn1p = jnp.concatenate(
        [xn1, jnp.zeros((n, _CHUNK - c_dim), jnp.float32)], axis=1)
    (out1a, out1b), s_t1 = _agnn_layer(
        xn1p, nrm1, src_p, dst_p, dst2, beta1, n, e_pad, sp, np_acc, "edge")
    return _final(out1a[:, :c_dim], out1b[:, :c_dim], s_t1, block)


# speedup vs baseline: 5.5137x; 1.7280x over previous
import jax
import jax.numpy as jnp
from jax import lax
from jax.experimental import pallas as pl
from jax.experimental.pallas import tpu as pltpu
from jax.experimental.pallas import tpu_sc as plsc

_LANES = 16
_CHUNK = 128
_N_TILES = 32


def _norm_body(x_ref, xn_ref, nrm_ref):
    x = x_ref[...]
    nrm = jnp.sqrt(jnp.sum(x * x, axis=1, keepdims=True))
    xn_ref[...] = x / (nrm + 1e-12)
    nrm_ref[...] = nrm


def _normalize(x, block_rows):
    n, f = x.shape
    return pl.pallas_call(
        _norm_body,
        grid=(n // block_rows,),
        in_specs=[pl.BlockSpec((block_rows, f), lambda i: (i, 0))],
        out_specs=[
            pl.BlockSpec((block_rows, f), lambda i: (i, 0)),
            pl.BlockSpec((block_rows, 1), lambda i: (i, 0)),
        ],
        out_shape=[
            jax.ShapeDtypeStruct((n, f), jnp.float32),
            jax.ShapeDtypeStruct((n, 1), jnp.float32),
        ],
    )(x)


def _mid_body(out0_ref, s_ref, w_ref, xn1_ref, nrm1_ref):
    s = jnp.sum(s_ref[...], axis=1, keepdims=True) + 1e-12
    h = jnp.tanh(out0_ref[...] / s)
    y = jnp.dot(h, w_ref[...], preferred_element_type=jnp.float32)
    n1 = jnp.sqrt(jnp.sum(y * y, axis=1, keepdims=True))
    xn1_ref[...] = y / (n1 + 1e-12)
    nrm1_ref[...] = n1


def _mid(out0, s_t, w1, block_rows):
    n, f = out0.shape
    t = s_t.shape[1]
    c = w1.shape[1]
    return pl.pallas_call(
        _mid_body,
        grid=(n // block_rows,),
        in_specs=[
            pl.BlockSpec((block_rows, f), lambda i: (i, 0)),
            pl.BlockSpec((block_rows, t), lambda i: (i, 0)),
            pl.BlockSpec((f, c), lambda i: (0, 0)),
        ],
        out_specs=[
            pl.BlockSpec((block_rows, c), lambda i: (i, 0)),
            pl.BlockSpec((block_rows, 1), lambda i: (i, 0)),
        ],
        out_shape=[
            jax.ShapeDtypeStruct((n, c), jnp.float32),
            jax.ShapeDtypeStruct((n, 1), jnp.float32),
        ],
    )(out0, s_t, w1)


def _fin_body(outa_ref, outb_ref, o_ref):
    a = outa_ref[...]
    b = outb_ref[...]
    c = o_ref.shape[1]
    s = a[:, c:c + 1] + b[:, c:c + 1] + 1e-12
    o_ref[...] = (a[:, :c] + b[:, :c]) / s


def _final(outa, outb, c_dim, block_rows):
    n, fw = outa.shape
    return pl.pallas_call(
        _fin_body,
        grid=(n // block_rows,),
        in_specs=[
            pl.BlockSpec((block_rows, fw), lambda i: (i, 0)),
            pl.BlockSpec((block_rows, fw), lambda i: (i, 0)),
        ],
        out_specs=pl.BlockSpec((block_rows, c_dim), lambda i: (i, 0)),
        out_shape=jax.ShapeDtypeStruct((n, c_dim), jnp.float32),
    )(outa, outb)


def _edge_weights(f_dim, e_pad, sp):
    ck = 64
    n_chunks = e_pad // ck
    chunks_per_tile = n_chunks // _N_TILES
    assert chunks_per_tile % 2 == 0
    tile_edges = chunks_per_tile * ck
    kf = f_dim // _LANES
    mesh = plsc.VectorSubcoreMesh(core_axis_name="c", subcore_axis_name="s")

    def body(xn_ref, scale_ref, src_ref, dst_ref, beta_ref, m_ref, spart_ref,
             src_a, src_b, dst_a, dst_b, rs_a, rs_b, rd_a, rd_b,
             cosb, m_all, s_loc, scale_v, beta_v, sem_a, sem_b):
        srcb = (src_a, src_b)
        dstb = (dst_a, dst_b)
        rsb = (rs_a, rs_b)
        rdb = (rd_a, rd_b)
        semb = (sem_a, sem_b)
        wid = lax.axis_index("s") * 2 + lax.axis_index("c")
        base = wid * chunks_per_tile
        pltpu.sync_copy(beta_ref, beta_v)
        pltpu.sync_copy(scale_ref, scale_v)
        beta_vec = beta_v[...]
        zeros16 = jnp.zeros((_LANES,), jnp.float32)
        lane0 = lax.iota(jnp.int32, _LANES) == 0

        def zero_step(i, carry):
            s_loc[pl.ds(i * _LANES, _LANES)] = zeros16
            return carry

        lax.fori_loop(0, sp // _LANES, zero_step, 0)

        def stage_a(j, b):
            pltpu.sync_copy(src_ref.at[pl.ds((base + j) * ck, ck)], srcb[b])
            pltpu.sync_copy(dst_ref.at[pl.ds((base + j) * ck, ck)], dstb[b])
            pltpu.async_copy(xn_ref.at[srcb[b]], rsb[b], semb[b])
            pltpu.async_copy(xn_ref.at[dstb[b]], rdb[b], semb[b])

        stage_a(0, 0)

        def pair_step(jj, carry):
            for b in range(2):
                j = jj * 2 + b
                nb = 1 - b

                @pl.when(j + 1 < chunks_per_tile)
                def _():
                    stage_a(j + 1, nb)

                pltpu.make_async_copy(xn_ref.at[srcb[b]], rsb[b],
                                      semb[b]).wait()
                pltpu.make_async_copy(xn_ref.at[dstb[b]], rdb[b],
                                      semb[b]).wait()
                rows_s = rsb[b]
                rows_d = rdb[b]

                def edge_step(e, ecarry):
                    acc = (rows_s[e, pl.ds(0, _LANES)] *
                           rows_d[e, pl.ds(0, _LANES)])
                    for k in range(1, kf):
                        sl = pl.ds(k * _LANES, _LANES)
                        acc = acc + rows_s[e, sl] * rows_d[e, sl]
                    cval = jnp.broadcast_to(jnp.sum(acc), (_LANES,))
                    eidx = jnp.broadcast_to(e, (_LANES,)).astype(jnp.int32)
                    plsc.store_scatter(cosb, [eidx], cval, mask=lane0)
                    return ecarry

                lax.fori_loop(0, ck, edge_step, 0)

                for g in range(ck // _LANES):
                    sl = pl.ds(g * _LANES, _LANES)
                    w_v = jnp.exp(beta_vec * cosb[sl])
                    sc = plsc.load_gather(scale_v, [srcb[b][sl]])
                    m_all[pl.ds(j * ck + g * _LANES, _LANES)] = w_v * sc
                    plsc.addupdate_scatter(s_loc, [dstb[b][sl]], w_v)
            return carry

        lax.fori_loop(0, chunks_per_tile // 2, pair_step, 0)
        pltpu.sync_copy(m_all, m_ref.at[pl.ds(base * ck, tile_edges)])
        pltpu.sync_copy(s_loc, spart_ref.at[wid])

    return pl.kernel(
        body,
        out_type=(
            jax.ShapeDtypeStruct((e_pad,), jnp.float32),
            jax.ShapeDtypeStruct((_N_TILES, sp), jnp.float32),
        ),
        mesh=mesh,
        compiler_params=pltpu.CompilerParams(needs_layout_passes=False),
        scratch_types=(
            pltpu.VMEM((ck,), jnp.int32),
            pltpu.VMEM((ck,), jnp.int32),
            pltpu.VMEM((ck,), jnp.int32),
            pltpu.VMEM((ck,), jnp.int32),
            pltpu.VMEM((ck, f_dim), jnp.float32),
            pltpu.VMEM((ck, f_dim), jnp.float32),
            pltpu.VMEM((ck, f_dim), jnp.float32),
            pltpu.VMEM((ck, f_dim), jnp.float32),
            pltpu.VMEM((ck,), jnp.float32),
            pltpu.VMEM((tile_edges,), jnp.float32),
            pltpu.VMEM((sp,), jnp.float32),
            pltpu.VMEM((sp,), jnp.float32),
            pltpu.VMEM((_LANES,), jnp.float32),
            pltpu.SemaphoreType.DMA,
            pltpu.SemaphoreType.DMA,
        ),
    )


def _message_scatter(fh, e_pad, np_acc, n_nodes, split):
    n_rows = e_pad // _CHUNK
    cpt = n_rows // (16 if split == "feat" else _N_TILES)
    assert cpt % 2 == 0
    jf = fh // _LANES
    mesh = plsc.VectorSubcoreMesh(core_axis_name="c", subcore_axis_name="s")

    def body(xcat_ref, src_ref, dst2_ref, m_ref, out_ref,
             src_a, src_b, di_a, di_b, m_a, m_b, rows_a, rows_b, zer, acc,
             gsem_a, gsem_b, ssem_a, ssem_b):
        srcb = (src_a, src_b)
        dib = (di_a, di_b)
        mb = (m_a, m_b)
        rowsb = (rows_a, rows_b)
        gsemb = (gsem_a, gsem_b)
        ssemb = (ssem_a, ssem_b)
        c = lax.axis_index("c")
        s_id = lax.axis_index("s")
        zeros16 = jnp.zeros((_LANES,), jnp.float32)
        if split == "feat":
            rbase = s_id * cpt
        else:
            rbase = (s_id * 2 + c) * cpt

        def zrow(e, carry):
            for j in range(jf):
                zer[e, pl.ds(j * _LANES, _LANES)] = zeros16
            return carry

        lax.fori_loop(0, _CHUNK, zrow, 0)

        rows_zero_per_tile = np_acc // 16
        nz = rows_zero_per_tile // _CHUNK
        zrem = rows_zero_per_tile % _CHUNK
        zbase = s_id * rows_zero_per_tile

        def zcopy(i, carry):
            pltpu.sync_copy(zer, acc.at[pl.ds(zbase + i * _CHUNK, _CHUNK)])
            return carry

        lax.fori_loop(0, nz, zcopy, 0)
        if zrem:
            pltpu.sync_copy(zer.at[pl.ds(0, zrem)],
                            acc.at[pl.ds(zbase + nz * _CHUNK, zrem)])
        plsc.subcore_barrier()

        def stage_a(j, b):
            r = rbase + j
            pltpu.sync_copy(src_ref.at[pl.ds(r * _CHUNK, _CHUNK)], srcb[b])
            pltpu.sync_copy(dst2_ref.at[pl.ds(r, 1)], dib[b])
            pltpu.sync_copy(m_ref.at[pl.ds(r * _CHUNK, _CHUNK)], mb[b])
            if split == "feat":
                coff = c * n_nodes
                for g in range(_CHUNK // _LANES):
                    sl = pl.ds(g * _LANES, _LANES)
                    srcb[b][sl] = srcb[b][sl] + coff
            pltpu.async_copy(xcat_ref.at[srcb[b]], rowsb[b], gsemb[b])

        stage_a(0, 0)

        def pair_step(jj, carry):
            for b in range(2):
                j = jj * 2 + b
                nb = 1 - b

                @pl.when((j + 1 < cpt) & (j >= 1))
                def _():
                    pltpu.make_async_copy(rowsb[nb], acc.at[dib[nb].at[0]],
                                          ssemb[nb]).wait()

                @pl.when(j + 1 < cpt)
                def _():
                    stage_a(j + 1, nb)

                pltpu.make_async_copy(xcat_ref.at[srcb[b]], rowsb[b],
                                      gsemb[b]).wait()
                rows = rowsb[b]
                m_v = mb[b]

                def scale_e(e, ecarry):
                    eidx = jnp.broadcast_to(e, (_LANES,)).astype(jnp.int32)
                    mv = plsc.load_gather(m_v, [eidx])
                    for j2 in range(jf):
                        sl = pl.ds(j2 * _LANES, _LANES)
                        rows[e, sl] = rows[e, sl] * mv
                    return ecarry

                lax.fori_loop(0, _CHUNK, scale_e, 0)
                pltpu.async_copy(rowsb[b], acc.at[dib[b].at[0]], ssemb[b],
                                 add=True)
            return carry

        lax.fori_loop(0, cpt // 2, pair_step, 0)
        pltpu.make_async_copy(rowsb[0], acc.at[dib[0].at[0]], ssemb[0]).wait()
        pltpu.make_async_copy(rowsb[1], acc.at[dib[1].at[0]], ssemb[1]).wait()
        plsc.subcore_barrier()

        obase = c * np_acc + zbase

        def ocopy(i, carry):
            pltpu.sync_copy(acc.at[pl.ds(zbase + i * _CHUNK, _CHUNK)],
                            out_ref.at[pl.ds(obase + i * _CHUNK, _CHUNK)])
            return carry

        lax.fori_loop(0, nz, ocopy, 0)
        if zrem:
            pltpu.sync_copy(acc.at[pl.ds(zbase + nz * _CHUNK, zrem)],
                            out_ref.at[pl.ds(obase + nz * _CHUNK, zrem)])

    return pl.kernel(
        body,
        out_type=jax.ShapeDtypeStruct((2 * np_acc, fh), jnp.float32),
        mesh=mesh,
        compiler_params=pltpu.CompilerParams(needs_layout_passes=False),
        scratch_types=(
            pltpu.VMEM((_CHUNK,), jnp.int32),
            pltpu.VMEM((_CHUNK,), jnp.int32),
            pltpu.VMEM((1, _CHUNK), jnp.int32),
            pltpu.VMEM((1, _CHUNK), jnp.int32),
            pltpu.VMEM((_CHUNK,), jnp.float32),
            pltpu.VMEM((_CHUNK,), jnp.float32),
            pltpu.VMEM((_CHUNK, fh), jnp.float32),
            pltpu.VMEM((_CHUNK, fh), jnp.float32),
            pltpu.VMEM((_CHUNK, fh), jnp.float32),
            pltpu.VMEM_SHARED((np_acc, fh), jnp.float32),
            pltpu.SemaphoreType.DMA,
            pltpu.SemaphoreType.DMA,
            pltpu.SemaphoreType.DMA,
            pltpu.SemaphoreType.DMA,
        ),
    )


def _fused_layer(e_pad, sp, np_acc):
    fw = _CHUNK
    jf = fw // _LANES
    n_rows = e_pad // _CHUNK
    cpt = n_rows // _N_TILES
    assert cpt % 2 == 0
    mesh = plsc.VectorSubcoreMesh(core_axis_name="c", subcore_axis_name="s")

    def body(xn_ref, scale_ref, src_ref, dst2_ref, beta_ref, out_ref,
             src_a, src_b, di_a, di_b, rs_a, rs_b, rd_a, rd_b,
             cosb, m_v, w_v, scale_v, beta_v, zer, acc,
             gsem_a, gsem_b, ssem_a, ssem_b):
        srcb = (src_a, src_b)
        dib = (di_a, di_b)
        rsb = (rs_a, rs_b)
        rdb = (rd_a, rd_b)
        gsemb = (gsem_a, gsem_b)
        ssemb = (ssem_a, ssem_b)
        c = lax.axis_index("c")
        s_id = lax.axis_index("s")
        wid = s_id * 2 + c
        rbase = wid * cpt
        pltpu.sync_copy(beta_ref, beta_v)
        pltpu.sync_copy(scale_ref, scale_v)
        beta_vec = beta_v[...]
        zeros16 = jnp.zeros((_LANES,), jnp.float32)
        lane0 = lax.iota(jnp.int32, _LANES) == 0
        e0vec = jnp.where(lane0, 1.0, 0.0).astype(jnp.float32)

        def zrow(e, carry):
            for j in range(jf):
                zer[e, pl.ds(j * _LANES, _LANES)] = zeros16
            return carry

        lax.fori_loop(0, _CHUNK, zrow, 0)

        rows_zero_per_tile = np_acc // 16
        nz = rows_zero_per_tile // _CHUNK
        zrem = rows_zero_per_tile % _CHUNK
        zbase = s_id * rows_zero_per_tile

        def zcopy(i, carry):
            pltpu.sync_copy(zer, acc.at[pl.ds(zbase + i * _CHUNK, _CHUNK)])
            return carry

        lax.fori_loop(0, nz, zcopy, 0)
        if zrem:
            pltpu.sync_copy(zer.at[pl.ds(0, zrem)],
                            acc.at[pl.ds(zbase + nz * _CHUNK, zrem)])
        plsc.subcore_barrier()

        def stage_a(j, b):
            r = rbase + j
            pltpu.sync_copy(src_ref.at[pl.ds(r * _CHUNK, _CHUNK)], srcb[b])
            pltpu.sync_copy(dst2_ref.at[pl.ds(r, 1)], dib[b])
            pltpu.async_copy(xn_ref.at[srcb[b]], rsb[b], gsemb[b])
            pltpu.async_copy(xn_ref.at[dib[b].at[0]], rdb[b], gsemb[b])

        stage_a(0, 0)

        def pair_step(jj, carry):
            for b in range(2):
                j = jj * 2 + b
                nb = 1 - b

                @pl.when((j + 1 < cpt) & (j >= 1))
                def _():
                    pltpu.make_async_copy(rsb[nb], acc.at[dib[nb].at[0]],
                                          ssemb[nb]).wait()

                @pl.when(j + 1 < cpt)
                def _():
                    stage_a(j + 1, nb)

                pltpu.make_async_copy(xn_ref.at[srcb[b]], rsb[b],
                                      gsemb[b]).wait()
                pltpu.make_async_copy(xn_ref.at[dib[b].at[0]], rdb[b],
                                      gsemb[b]).wait()
                rows_s = rsb[b]
                rows_d = rdb[b]

                def edge_step(e, ecarry):
                    acc_v = (rows_s[e, pl.ds(0, _LANES)] *
                             rows_d[e, pl.ds(0, _LANES)])
                    for k in range(1, 4):
                        sl = pl.ds(k * _LANES, _LANES)
                        acc_v = acc_v + rows_s[e, sl] * rows_d[e, sl]
                    cval = jnp.broadcast_to(jnp.sum(acc_v), (_LANES,))
                    eidx = jnp.broadcast_to(e, (_LANES,)).astype(jnp.int32)
                    plsc.store_scatter(cosb, [eidx], cval, mask=lane0)
                    return ecarry

                lax.fori_loop(0, _CHUNK, edge_step, 0)

                for g in range(_CHUNK // _LANES):
                    sl = pl.ds(g * _LANES, _LANES)
                    wv = jnp.exp(beta_vec * cosb[sl])
                    sc = plsc.load_gather(scale_v, [srcb[b][sl]])
                    w_v[sl] = wv
                    m_v[sl] = wv * sc

                def scale_e(e, ecarry):
                    eidx = jnp.broadcast_to(e, (_LANES,)).astype(jnp.int32)
                    mv = plsc.load_gather(m_v, [eidx])
                    for j2 in range(4):
                        sl = pl.ds(j2 * _LANES, _LANES)
                        rows_s[e, sl] = rows_s[e, sl] * mv
                    wv = plsc.load_gather(w_v, [eidx])
                    rows_s[e, pl.ds(4 * _LANES, _LANES)] = wv * e0vec
                    return ecarry

                lax.fori_loop(0, _CHUNK, scale_e, 0)
                pltpu.async_copy(rsb[b], acc.at[dib[b].at[0]], ssemb[b],
                                 add=True)
            return carry

        lax.fori_loop(0, cpt // 2, pair_step, 0)
        pltpu.make_async_copy(rsb[0], acc.at[dib[0].at[0]], ssemb[0]).wait()
        pltpu.make_async_copy(rsb[1], acc.at[dib[1].at[0]], ssemb[1]).wait()
        plsc.subcore_barrier()

        obase = c * np_acc + zbase

        def ocopy(i, carry):
            pltpu.sync_copy(acc.at[pl.ds(zbase + i * _CHUNK, _CHUNK)],
                            out_ref.at[pl.ds(obase + i * _CHUNK, _CHUNK)])
            return carry

        lax.fori_loop(0, nz, ocopy, 0)
        if zrem:
            pltpu.sync_copy(acc.at[pl.ds(zbase + nz * _CHUNK, zrem)],
                            out_ref.at[pl.ds(obase + nz * _CHUNK, zrem)])

    return pl.kernel(
        body,
        out_type=jax.ShapeDtypeStruct((2 * np_acc, fw), jnp.float32),
        mesh=mesh,
        compiler_params=pltpu.CompilerParams(needs_layout_passes=False),
        scratch_types=(
            pltpu.VMEM((_CHUNK,), jnp.int32),
            pltpu.VMEM((_CHUNK,), jnp.int32),
            pltpu.VMEM((1, _CHUNK), jnp.int32),
            pltpu.VMEM((1, _CHUNK), jnp.int32),
            pltpu.VMEM((_CHUNK, _CHUNK), jnp.float32),
            pltpu.VMEM((_CHUNK, _CHUNK), jnp.float32),
            pltpu.VMEM((_CHUNK, _CHUNK), jnp.float32),
            pltpu.VMEM((_CHUNK, _CHUNK), jnp.float32),
            pltpu.VMEM((_CHUNK,), jnp.float32),
            pltpu.VMEM((_CHUNK,), jnp.float32),
            pltpu.VMEM((_CHUNK,), jnp.float32),
            pltpu.VMEM((sp,), jnp.float32),
            pltpu.VMEM((_LANES,), jnp.float32),
            pltpu.VMEM((_CHUNK, _CHUNK), jnp.float32),
            pltpu.VMEM_SHARED((np_acc, _CHUNK), jnp.float32),
            pltpu.SemaphoreType.DMA,
            pltpu.SemaphoreType.DMA,
            pltpu.SemaphoreType.DMA,
            pltpu.SemaphoreType.DMA,
        ),
    )


def _fin2_body(outa_ref, outb_ref, s_ref, o_ref):
    s = jnp.sum(s_ref[...], axis=1, keepdims=True) + 1e-12
    o_ref[...] = (outa_ref[...] + outb_ref[...]) / s


def _final2(outa, outb, s_t, block_rows):
    n, c = outa.shape
    t = s_t.shape[1]
    return pl.pallas_call(
        _fin2_body,
        grid=(n // block_rows,),
        in_specs=[
            pl.BlockSpec((block_rows, c), lambda i: (i, 0)),
            pl.BlockSpec((block_rows, c), lambda i: (i, 0)),
            pl.BlockSpec((block_rows, t), lambda i: (i, 0)),
        ],
        out_specs=pl.BlockSpec((block_rows, c), lambda i: (i, 0)),
        out_shape=jax.ShapeDtypeStruct((n, c), jnp.float32),
    )(outa, outb, s_t)


def _agnn_layer(xn, nrm, src_p, dst_p, dst2, beta, n, e_pad, sp, np_acc,
                split):
    f = xn.shape[1]
    xn_pad = jnp.concatenate([xn, jnp.zeros((1, f), jnp.float32)])
    scale_pad = jnp.concatenate(
        [nrm[:, 0], jnp.zeros((sp - n,), jnp.float32)])
    beta_arr = jnp.zeros((_LANES,), jnp.float32) + beta.astype(jnp.float32)
    m_e, s_part = _edge_weights(f, e_pad, sp)(
        xn_pad, scale_pad, src_p, dst_p, beta_arr)
    s_t = s_part[:, :n].T
    if split == "feat":
        xcat = jnp.concatenate([xn[:, : f // 2], xn[:, f // 2:]], axis=0)
        out_cat = _message_scatter(f // 2, e_pad, np_acc, n, split)(
            xcat, src_p, dst2, m_e)
        out = jnp.concatenate(
            [out_cat[:n], out_cat[np_acc:np_acc + n]], axis=1)
        return (out,), s_t
    out_cat = _message_scatter(f, e_pad, np_acc, n, split)(
        xn, src_p, dst2, m_e)
    return (out_cat[:n], out_cat[np_acc:np_acc + n]), s_t


@jax.jit
def kernel(x, edge_index, W1, beta0, beta1):
    n, f = x.shape
    c_dim = W1.shape[1]
    e = edge_index.shape[1]
    e_pad = -(-e // (_N_TILES * _CHUNK)) * (_N_TILES * _CHUNK)
    sp = -(-(n + 1) // _LANES) * _LANES
    np_acc = -(-(n + 1) // (16 * 8)) * (16 * 8)
    block = 1000 if n % 1000 == 0 else 8

    src = edge_index[0].astype(jnp.int32)
    dst = edge_index[1].astype(jnp.int32)
    src_p = jnp.concatenate([src, jnp.zeros((e_pad - e,), jnp.int32)])
    dst_p = jnp.concatenate([dst, jnp.full((e_pad - e,), n, jnp.int32)])
    dst2 = dst_p.reshape(e_pad // _CHUNK, _CHUNK)

    xn, nrm = _normalize(x, block)
    (out0,), s_t0 = _agnn_layer(
        xn, nrm, src_p, dst_p, dst2, beta0, n, e_pad, sp, np_acc, "feat")

    xn1, nrm1 = _mid(out0, s_t0, W1, block)
    xn1p = jnp.concatenate(
        [xn1, jnp.zeros((n, _CHUNK - c_dim), jnp.float32)], axis=1)
    (out1a, out1b), s_t1 = _agnn_layer(
        xn1p, nrm1, src_p, dst_p, dst2, beta1, n, e_pad, sp, np_acc, "edge")
    return _final2(out1a[:, :c_dim], out1b[:, :c_dim], s_t1, block)
